# Initial kernel scaffold; baseline (speedup 1.0000x reference)
import functools

import jax
import jax.numpy as jnp
from jax import lax
from jax.experimental import pallas as pl
from jax.experimental.pallas import tpu as pltpu
from jax.experimental.pallas import tpu_sc as plsc

B = 16384
L = 50
D = 64
LANES = 16
NC = 2
NS = 16
NW = NC * NS
BPW = B // NW
CB = 8
NCHUNK = BPW // CB


def _make_kernel():
    mesh = plsc.VectorSubcoreMesh(core_axis_name="c", subcore_axis_name="s")

    @functools.partial(
        pl.kernel,
        mesh=mesh,
        out_type=jax.ShapeDtypeStruct((B, D), jnp.float32),
        scratch_types=[
            pltpu.VMEM((CB, L), jnp.int32),
            pltpu.VMEM((CB, L, D), jnp.float32),
            pltpu.VMEM((CB, D), jnp.float32),
            pltpu.SemaphoreType.DMA,
            pltpu.SemaphoreType.DMA,
        ],
    )
    def emb_mean(labels_hbm, table_hbm, out_hbm, idx_v, rows_v, out_v,
                 sem_g, sem_o):
        wid = lax.axis_index("s") * NC + lax.axis_index("c")
        base = wid * BPW

        def chunk_body(c, carry):
            row0 = base + c * CB
            pltpu.sync_copy(labels_hbm.at[pl.ds(row0, CB)], idx_v)
            copies = [
                pltpu.async_copy(table_hbm.at[idx_v.at[i]], rows_v.at[i],
                                 sem_g)
                for i in range(CB)
            ]
            for cp in copies:
                cp.wait()
            for i in range(CB):
                def l_body(l, acc, i=i):
                    return tuple(
                        acc[d] + rows_v[i, l, pl.ds(d * LANES, LANES)]
                        for d in range(D // LANES)
                    )
                acc0 = tuple(
                    jnp.zeros((LANES,), jnp.float32)
                    for _ in range(D // LANES)
                )
                acc = lax.fori_loop(0, L, l_body, acc0)
                for d in range(D // LANES):
                    out_v[i, pl.ds(d * LANES, LANES)] = acc[d] * (1.0 / L)
            pltpu.async_copy(out_v, out_hbm.at[pl.ds(row0, CB)], sem_o).wait()
            return carry

        lax.fori_loop(0, NCHUNK, chunk_body, 0)

    return emb_mean


_emb_mean = _make_kernel()


@jax.jit
def kernel(label_text, table):
    return _emb_mean(label_text, table)


# speedup vs baseline: 2.7833x; 2.7833x over previous
import functools

import jax
import jax.numpy as jnp
from jax import lax
from jax.experimental import pallas as pl
from jax.experimental.pallas import tpu as pltpu
from jax.experimental.pallas import tpu_sc as plsc

B = 16384
L = 50
D = 64
LANES = 16
ND = D // LANES
NC = 2
NS = 16
NW = NC * NS
BPW = B // NW
PAIRS = 2
CB = 4
ROWS_PER_CHUNK = CB * PAIRS
NCHUNK = BPW // ROWS_PER_CHUNK
LUNROLL = 10


def _make_kernel():
    mesh = plsc.VectorSubcoreMesh(core_axis_name="c", subcore_axis_name="s")

    @functools.partial(
        pl.kernel,
        mesh=mesh,
        compiler_params=pltpu.CompilerParams(use_tc_tiling_on_sc=False),
        out_type=jax.ShapeDtypeStruct((B, D), jnp.float32),
        scratch_types=[
            pltpu.VMEM((2, CB, PAIRS * L), jnp.int32),
            pltpu.VMEM((2, CB, PAIRS * L, D), jnp.float32),
            pltpu.VMEM((BPW, D), jnp.float32),
            pltpu.SemaphoreType.DMA,
            pltpu.SemaphoreType.DMA,
            pltpu.SemaphoreType.DMA,
            pltpu.SemaphoreType.DMA,
        ],
    )
    def emb_mean(labels_hbm, table_hbm, out_hbm, idx_v, rows_v, out_v,
                 sem_i0, sem_i1, sem_g0, sem_g1):
        wid = lax.axis_index("s") * NC + lax.axis_index("c")
        pair_base = wid * (BPW // PAIRS)
        sem_i = (sem_i0, sem_i1)
        sem_g = (sem_g0, sem_g1)

        def idx_copy(c, p):
            return pltpu.make_async_copy(
                labels_hbm.at[pl.ds(pair_base + c * CB, CB)],
                idx_v.at[p], sem_i[p])

        def gather_copy(p, i):
            return pltpu.make_async_copy(
                table_hbm.at[idx_v.at[p].at[i]],
                rows_v.at[p].at[i], sem_g[p])

        def fire_gathers(p):
            for i in range(CB):
                gather_copy(p, i).start()

        def wait_gathers(p):
            for i in range(CB):
                gather_copy(p, i).wait()

        def accumulate(c, p):
            for i in range(CB):
                for h in range(PAIRS):
                    jbase = h * L

                    def l_body(j, acc, i=i, jbase=jbase):
                        new = []
                        for d in range(ND):
                            a = acc[d]
                            for u in range(LUNROLL):
                                a = a + rows_v[p, i, jbase + j * LUNROLL + u,
                                               pl.ds(d * LANES, LANES)]
                            new.append(a)
                        return tuple(new)

                    acc0 = tuple(jnp.zeros((LANES,), jnp.float32)
                                 for _ in range(ND))
                    acc = lax.fori_loop(0, L // LUNROLL, l_body, acc0)
                    r = (c * CB + i) * PAIRS + h
                    for d in range(ND):
                        out_v[r, pl.ds(d * LANES, LANES)] = (
                            acc[d] * (1.0 / L))

        idx_copy(0, 0).start()
        idx_copy(1, 1).start()
        idx_copy(0, 0).wait()
        fire_gathers(0)

        def chunk_pair(c2, carry):
            for p in (0, 1):
                c = c2 * 2 + p
                idx_copy(c + 1, 1 - p).wait()
                fire_gathers(1 - p)
                wait_gathers(p)
                idx_copy(c + 2, p).start()
                accumulate(c, p)
            return carry

        lax.fori_loop(0, NCHUNK // 2 - 1, chunk_pair, 0)

        idx_copy(NCHUNK - 1, 1).wait()
        fire_gathers(1)
        wait_gathers(0)
        accumulate(NCHUNK - 2, 0)
        wait_gathers(1)
        accumulate(NCHUNK - 1, 1)

        pltpu.sync_copy(
            out_v, out_hbm.at[pl.ds(wid * BPW, BPW)])

    return emb_mean


_emb_mean = _make_kernel()


@jax.jit
def kernel(label_text, table):
    labels2 = label_text.reshape(B // PAIRS, PAIRS * L)
    return _emb_mean(labels2, table)
